# Initial kernel scaffold; baseline (speedup 1.0000x reference)
import functools

import jax
import jax.numpy as jnp
from jax import lax
from jax.experimental import pallas as pl
from jax.experimental.pallas import tpu as pltpu
from jax.experimental.pallas import tpu_sc as plsc

_NUM_LABELS = 1000
_D = 128
_L = 16
_NC, _NS = 2, 16
_NW = _NC * _NS
_CHUNK = 128


def kernel(vertex_ids, labels, degrees, vertex_embed, label_embed, deg_W, deg_b):
    n = vertex_ids.shape[0]
    b_per_w = n // _NW
    n_chunks = b_per_w // _CHUNK

    vertex_ids = vertex_ids.astype(jnp.int32)
    labels = labels.astype(jnp.int32)

    mesh = plsc.VectorSubcoreMesh(
        core_axis_name="c", subcore_axis_name="s",
        num_cores=_NC, num_subcores=_NS,
    )

    @functools.partial(
        pl.kernel,
        out_type=jax.ShapeDtypeStruct((n, _D), jnp.float32),
        mesh=mesh,
        scratch_types=[
            pltpu.VMEM((n_chunks, _CHUNK), jnp.int32),
            pltpu.VMEM((n_chunks, _CHUNK), jnp.int32),
            pltpu.VMEM((b_per_w,), jnp.float32),
            pltpu.VMEM((_D,), jnp.float32),
            pltpu.VMEM((_D,), jnp.float32),
            pltpu.VMEM((b_per_w, _D), jnp.float32),
            pltpu.SemaphoreType.DMA,
        ],
    )
    def run(vid_hbm, lbl_hbm, deg_hbm, vtab_hbm, ltab_hbm, w_hbm, b_hbm,
            out_hbm, vidx, lidx, degv, wv, bv, rows, sem):
        wid = lax.axis_index("s") * _NC + lax.axis_index("c")
        base = wid * b_per_w

        pltpu.sync_copy(
            vid_hbm.at[pl.ds(base, b_per_w)],
            vidx.at[:].reshape(b_per_w),
        )
        pltpu.sync_copy(
            lbl_hbm.at[pl.ds(base, b_per_w)],
            lidx.at[:].reshape(b_per_w),
        )
        pltpu.sync_copy(deg_hbm.at[pl.ds(base, b_per_w)], degv)
        pltpu.sync_copy(w_hbm, wv)
        pltpu.sync_copy(b_hbm, bv)

        for j in range(n_chunks):
            pltpu.async_copy(
                vtab_hbm.at[vidx.at[j]],
                rows.at[pl.ds(j * _CHUNK, _CHUNK)],
                sem,
            )

        def fix_labels(i, _):
            j = i // (_CHUNK // _L)
            o = (i % (_CHUNK // _L)) * _L
            lab = lidx[j, pl.ds(o, _L)]
            lab = jnp.where((lab >= _NUM_LABELS) | (lab == -1), _NUM_LABELS, lab)
            lidx[j, pl.ds(o, _L)] = jnp.clip(lab, 0, _NUM_LABELS)
            return 0

        lax.fori_loop(0, b_per_w // _L, fix_labels, 0)

        for j in range(n_chunks):
            pltpu.make_async_copy(
                vtab_hbm.at[vidx.at[j]],
                rows.at[pl.ds(j * _CHUNK, _CHUNK)],
                sem,
            ).wait()
        for j in range(n_chunks):
            pltpu.async_copy(
                ltab_hbm.at[lidx.at[j]],
                rows.at[pl.ds(j * _CHUNK, _CHUNK)],
                sem,
                add=True,
            )
        for j in range(n_chunks):
            pltpu.make_async_copy(
                ltab_hbm.at[lidx.at[j]],
                rows.at[pl.ds(j * _CHUNK, _CHUNK)],
                sem,
            ).wait()

        ws = [wv[pl.ds(j * _L, _L)] for j in range(_D // _L)]
        bs = [bv[pl.ds(j * _L, _L)] for j in range(_D // _L)]

        def row_update(i, _):
            d = plsc.load_gather(degv, [jnp.full((_L,), i, jnp.int32)])
            d = jnp.where(d * 0.0 == 0.0, d, 1.0)
            d = jnp.maximum(d, 1.0)
            for j in range(_D // _L):
                sl = pl.ds(j * _L, _L)
                rows[i, sl] = rows[i, sl] + d * ws[j] + bs[j]
            return 0

        lax.fori_loop(0, b_per_w, row_update, 0)

        pltpu.sync_copy(rows, out_hbm.at[pl.ds(base, b_per_w)])

    return run(vertex_ids, labels, degrees, vertex_embed, label_embed,
               deg_W, deg_b)


# speedup vs baseline: 2.2149x; 2.2149x over previous
import functools

import jax
import jax.numpy as jnp
from jax import lax
from jax.experimental import pallas as pl
from jax.experimental.pallas import tpu as pltpu
from jax.experimental.pallas import tpu_sc as plsc

_NUM_LABELS = 1000
_D = 128
_L = 16
_NC, _NS = 2, 16
_NW = _NC * _NS
_CHUNK = 128


def kernel(vertex_ids, labels, degrees, vertex_embed, label_embed, deg_W, deg_b):
    n = vertex_ids.shape[0]
    b_per_w = n // _NW
    n_chunks = b_per_w // _CHUNK

    vertex_ids = vertex_ids.astype(jnp.int32)
    labels = labels.astype(jnp.int32)

    mesh = plsc.VectorSubcoreMesh(
        core_axis_name="c", subcore_axis_name="s",
        num_cores=_NC, num_subcores=_NS,
    )

    @functools.partial(
        pl.kernel,
        out_type=jax.ShapeDtypeStruct((n, _D), jnp.float32),
        mesh=mesh,
        scratch_types=[
            pltpu.VMEM((n_chunks, _CHUNK), jnp.int32),
            pltpu.VMEM((n_chunks, _CHUNK), jnp.int32),
            pltpu.VMEM((b_per_w,), jnp.float32),
            pltpu.VMEM((_D,), jnp.float32),
            pltpu.VMEM((_D,), jnp.float32),
            pltpu.VMEM((b_per_w, _D), jnp.float32),
            pltpu.SemaphoreType.DMA,
        ],
    )
    def run(vid_hbm, lbl_hbm, deg_hbm, vtab_hbm, ltab_hbm, w_hbm, b_hbm,
            out_hbm, vidx, lidx, degv, wv, bv, rows, sem):
        wid = lax.axis_index("s") * _NC + lax.axis_index("c")
        base = wid * b_per_w

        for j in range(n_chunks):
            pltpu.sync_copy(vid_hbm.at[pl.ds(base + j * _CHUNK, _CHUNK)],
                            vidx.at[j])
            pltpu.sync_copy(lbl_hbm.at[pl.ds(base + j * _CHUNK, _CHUNK)],
                            lidx.at[j])
        pltpu.sync_copy(deg_hbm.at[pl.ds(base, b_per_w)], degv)
        pltpu.sync_copy(w_hbm, wv)
        pltpu.sync_copy(b_hbm, bv)

        for j in range(n_chunks):
            pltpu.async_copy(
                vtab_hbm.at[vidx.at[j]],
                rows.at[pl.ds(j * _CHUNK, _CHUNK)],
                sem,
            )

        def fix_labels(i, _):
            j = i // (_CHUNK // _L)
            o = (i % (_CHUNK // _L)) * _L
            lab = lidx[j, pl.ds(o, _L)]
            lab = jnp.where((lab >= _NUM_LABELS) | (lab == -1), _NUM_LABELS, lab)
            lidx[j, pl.ds(o, _L)] = jnp.clip(lab, 0, _NUM_LABELS)
            return 0

        lax.fori_loop(0, b_per_w // _L, fix_labels, 0)

        for j in range(n_chunks):
            pltpu.make_async_copy(
                vtab_hbm.at[vidx.at[j]],
                rows.at[pl.ds(j * _CHUNK, _CHUNK)],
                sem,
            ).wait()
        for j in range(n_chunks):
            pltpu.async_copy(
                ltab_hbm.at[lidx.at[j]],
                rows.at[pl.ds(j * _CHUNK, _CHUNK)],
                sem,
                add=True,
            )
        for j in range(n_chunks):
            pltpu.make_async_copy(
                ltab_hbm.at[lidx.at[j]],
                rows.at[pl.ds(j * _CHUNK, _CHUNK)],
                sem,
            ).wait()

        ws = [wv[pl.ds(j * _L, _L)] for j in range(_D // _L)]
        bs = [bv[pl.ds(j * _L, _L)] for j in range(_D // _L)]

        def group_update(g, _):
            d16 = degv[pl.ds(g * _L, _L)]
            d16 = jnp.where(d16 * 0.0 == 0.0, d16, 1.0)
            d16 = jnp.maximum(d16, 1.0)
            for k in range(_L):
                d = jnp.full((_L,), d16[k], jnp.float32)
                i = g * _L + k
                for j in range(_D // _L):
                    sl = pl.ds(j * _L, _L)
                    rows[i, sl] = rows[i, sl] + d * ws[j] + bs[j]
            return 0

        lax.fori_loop(0, b_per_w // _L, group_update, 0)

        pltpu.sync_copy(rows, out_hbm.at[pl.ds(base, b_per_w)])

    return run(vertex_ids, labels, degrees, vertex_embed, label_embed,
               deg_W, deg_b)


# speedup vs baseline: 2.4139x; 1.0899x over previous
import functools

import jax
import jax.numpy as jnp
from jax import lax
from jax.experimental import pallas as pl
from jax.experimental.pallas import tpu as pltpu
from jax.experimental.pallas import tpu_sc as plsc

_NUM_LABELS = 1000
_D = 128
_L = 16
_NC, _NS = 2, 16
_NW = _NC * _NS
_CHUNK = 128


def kernel(vertex_ids, labels, degrees, vertex_embed, label_embed, deg_W, deg_b):
    n = vertex_ids.shape[0]
    b_per_w = n // _NW
    n_chunks = b_per_w // _CHUNK
    gpc = _CHUNK // _L

    vertex_ids = vertex_ids.astype(jnp.int32)
    labels = labels.astype(jnp.int32)

    mesh = plsc.VectorSubcoreMesh(
        core_axis_name="c", subcore_axis_name="s",
        num_cores=_NC, num_subcores=_NS,
    )

    @functools.partial(
        pl.kernel,
        out_type=jax.ShapeDtypeStruct((n, _D), jnp.float32),
        mesh=mesh,
        scratch_types=[
            pltpu.VMEM((n_chunks, _CHUNK), jnp.int32),
            pltpu.VMEM((n_chunks, _CHUNK), jnp.int32),
            pltpu.VMEM((b_per_w,), jnp.float32),
            pltpu.VMEM((_D,), jnp.float32),
            pltpu.VMEM((_D,), jnp.float32),
            pltpu.VMEM((b_per_w, _D), jnp.float32),
            pltpu.SemaphoreType.DMA,
            pltpu.SemaphoreType.DMA,
        ] + [pltpu.SemaphoreType.DMA] * n_chunks,
    )
    def run(vid_hbm, lbl_hbm, deg_hbm, vtab_hbm, ltab_hbm, w_hbm, b_hbm,
            out_hbm, vidx, lidx, degv, wv, bv, rows, sem_in, sem_wb, *csem):
        wid = lax.axis_index("s") * _NC + lax.axis_index("c")
        base = wid * b_per_w

        ins = []
        for j in range(n_chunks):
            ins.append(pltpu.async_copy(
                vid_hbm.at[pl.ds(base + j * _CHUNK, _CHUNK)], vidx.at[j],
                sem_in))
            ins.append(pltpu.async_copy(
                lbl_hbm.at[pl.ds(base + j * _CHUNK, _CHUNK)], lidx.at[j],
                sem_in))
        ins.append(pltpu.async_copy(deg_hbm.at[pl.ds(base, b_per_w)], degv,
                                    sem_in))
        ins.append(pltpu.async_copy(w_hbm, wv, sem_in))
        ins.append(pltpu.async_copy(b_hbm, bv, sem_in))
        for c in ins:
            c.wait()

        ws = [wv[pl.ds(j * _L, _L)] for j in range(_D // _L)]
        bs = [bv[pl.ds(j * _L, _L)] for j in range(_D // _L)]

        def compute_chunk(j):
            for i in range(gpc):
                lab = lidx[j, pl.ds(i * _L, _L)]
                lab = jnp.where((lab >= _NUM_LABELS) | (lab == -1),
                                _NUM_LABELS, lab)
                lidx[j, pl.ds(i * _L, _L)] = jnp.clip(lab, 0, _NUM_LABELS)

            def group_update(g, _):
                d16 = degv[pl.ds(g * _L, _L)]
                d16 = jnp.where(d16 * 0.0 == 0.0, d16, 1.0)
                d16 = jnp.maximum(d16, 1.0)
                for k in range(_L):
                    d = jnp.full((_L,), d16[k], jnp.float32)
                    i = g * _L + k
                    for q in range(_D // _L):
                        rows[i, pl.ds(q * _L, _L)] = d * ws[q] + bs[q]
                return 0

            lax.fori_loop(j * gpc, (j + 1) * gpc, group_update, 0)

        adds = []
        for j in range(n_chunks):
            compute_chunk(j)
            dst = rows.at[pl.ds(j * _CHUNK, _CHUNK)]
            adds.append((
                pltpu.async_copy(vtab_hbm.at[vidx.at[j]], dst, csem[j],
                                 add=True),
                pltpu.async_copy(ltab_hbm.at[lidx.at[j]], dst, csem[j],
                                 add=True),
            ))

        wbs = []
        for j in range(n_chunks):
            adds[j][0].wait()
            adds[j][1].wait()
            wbs.append(pltpu.async_copy(
                rows.at[pl.ds(j * _CHUNK, _CHUNK)],
                out_hbm.at[pl.ds(base + j * _CHUNK, _CHUNK)],
                sem_wb))
        for w in wbs:
            w.wait()

    return run(vertex_ids, labels, degrees, vertex_embed, label_embed,
               deg_W, deg_b)


# speedup vs baseline: 2.6651x; 1.1040x over previous
import functools

import jax
import jax.numpy as jnp
from jax import lax
from jax.experimental import pallas as pl
from jax.experimental.pallas import tpu as pltpu
from jax.experimental.pallas import tpu_sc as plsc

_NUM_LABELS = 1000
_D = 128
_L = 16
_NC, _NS = 2, 16
_NW = _NC * _NS
_CHUNK = 128


def kernel(vertex_ids, labels, degrees, vertex_embed, label_embed, deg_W, deg_b):
    n = vertex_ids.shape[0]
    b_per_w = n // _NW
    n_chunks = b_per_w // _CHUNK
    gpc = _CHUNK // _L

    vertex_ids = vertex_ids.astype(jnp.int32)
    labels = labels.astype(jnp.int32)

    mesh = plsc.VectorSubcoreMesh(
        core_axis_name="c", subcore_axis_name="s",
        num_cores=_NC, num_subcores=_NS,
    )

    @functools.partial(
        pl.kernel,
        out_type=jax.ShapeDtypeStruct((n, _D), jnp.float32),
        mesh=mesh,
        scratch_types=[
            pltpu.VMEM((n_chunks, _CHUNK), jnp.int32),
            pltpu.VMEM((n_chunks, _CHUNK), jnp.int32),
            pltpu.VMEM((b_per_w,), jnp.float32),
            pltpu.VMEM((_D,), jnp.float32),
            pltpu.VMEM((_D,), jnp.float32),
            pltpu.VMEM((b_per_w, _D), jnp.float32),
            pltpu.SemaphoreType.DMA,
            pltpu.SemaphoreType.DMA,
        ],
    )
    def run(vid_hbm, lbl_hbm, deg_hbm, vtab_hbm, ltab_hbm, w_hbm, b_hbm,
            out_hbm, vidx, lidx, degv, wv, bv, rows, sem_in, sem_add):
        wid = lax.axis_index("s") * _NC + lax.axis_index("c")
        base = wid * b_per_w

        ins = []
        for j in range(n_chunks):
            ins.append(pltpu.async_copy(
                vid_hbm.at[pl.ds(base + j * _CHUNK, _CHUNK)], vidx.at[j],
                sem_in))
            ins.append(pltpu.async_copy(
                lbl_hbm.at[pl.ds(base + j * _CHUNK, _CHUNK)], lidx.at[j],
                sem_in))
        ins.append(pltpu.async_copy(deg_hbm.at[pl.ds(base, b_per_w)], degv,
                                    sem_in))
        ins.append(pltpu.async_copy(w_hbm, wv, sem_in))
        ins.append(pltpu.async_copy(b_hbm, bv, sem_in))
        for c in ins:
            c.wait()

        def fix_labels(i, _):
            j = i // gpc
            o = (i % gpc) * _L
            lab = lidx[j, pl.ds(o, _L)]
            lab = jnp.where((lab >= _NUM_LABELS) | (lab == -1),
                            _NUM_LABELS, lab)
            lidx[j, pl.ds(o, _L)] = jnp.clip(lab, 0, _NUM_LABELS)
            return 0

        lax.fori_loop(0, b_per_w // _L, fix_labels, 0)

        ws = [wv[pl.ds(q * _L, _L)] for q in range(_D // _L)]
        bs = [bv[pl.ds(q * _L, _L)] for q in range(_D // _L)]

        def group_update(g, _):
            d16 = degv[pl.ds(g * _L, _L)]
            d16 = jnp.where(d16 * 0.0 == 0.0, d16, 1.0)
            d16 = jnp.maximum(d16, 1.0)
            for k in range(_L):
                d = jnp.full((_L,), d16[k], jnp.float32)
                i = g * _L + k
                for q in range(_D // _L):
                    rows[i, pl.ds(q * _L, _L)] = d * ws[q] + bs[q]
            return 0

        def chunk_body(j, _):
            lax.fori_loop(j * gpc, (j + 1) * gpc, group_update, 0)
            dst = rows.at[pl.ds(j * _CHUNK, _CHUNK)]
            pltpu.async_copy(vtab_hbm.at[vidx.at[j]], dst, sem_add, add=True)
            pltpu.async_copy(ltab_hbm.at[lidx.at[j]], dst, sem_add, add=True)
            return 0

        lax.fori_loop(0, n_chunks, chunk_body, 0)

        def drain_body(j, _):
            dst = rows.at[pl.ds(j * _CHUNK, _CHUNK)]
            pltpu.make_async_copy(vtab_hbm.at[vidx.at[j]], dst,
                                  sem_add).wait()
            pltpu.make_async_copy(ltab_hbm.at[lidx.at[j]], dst,
                                  sem_add).wait()
            return 0

        lax.fori_loop(0, n_chunks, drain_body, 0)

        pltpu.sync_copy(rows, out_hbm.at[pl.ds(base, b_per_w)])

    return run(vertex_ids, labels, degrees, vertex_embed, label_embed,
               deg_W, deg_b)
